# baseline (speedup 1.0000x reference)
import functools

import jax
import jax.numpy as jnp
from jax.experimental import pallas as pl
from jax.experimental.pallas import tpu as pltpu

_CH = 8
_PAD = 1


def _geometry(t_shapes):
    plan = []
    cin = 1
    for (k, wpcin, wocout) in t_shapes:
        wp = wpcin // cin
        wo = wocout // _CH
        win = wp - 2 * _PAD
        ho = wp - k + 1
        assert ho == wo
        plan.append(dict(k=k, cin=cin, win=win, hin=win, wp=wp, hp=wp,
                         ho=ho, wo=wo))
        cin = _CH
    return plan


def _slab_width(g, li):
    K = g["win"] * g["cin"]
    if li == 0:
        return K, K
    return K, -(-K // 128) * 128


def _fused_kernel(x1_ref, x2_ref, w0, w1, w2, w3, b0, b1, b2, b3,
                  lw, lb, ow, ob, o_ref, il, *, plan, batch):
    B = batch
    bh = B // 2
    ws = (w0, w1, w2, w3)
    bs = (b0, b1, b2, b3)

    act = None
    prev_rows = plan[0]["hin"]
    for li, g in enumerate(plan):
        k, ho = g["k"], g["ho"]
        K, kp = _slab_width(g, li)
        zgap = (jnp.zeros((ho, B, kp - K), il.dtype) if kp > K else None)
        for i in range(k):
            col = i * kp
            lo = max(0, 1 - i)
            hi = min(ho, prev_rows + 1 - i)
            if li == 0:
                il[lo:hi, 0:bh, col:col + K] = x1_ref[i + lo - 1:i + hi - 1]
                il[lo:hi, bh:B, col:col + K] = x2_ref[i + lo - 1:i + hi - 1]
            else:
                il[lo:hi, :, col:col + K] = act[i + lo - 1:i + hi - 1]
            if zgap is not None:
                il[lo:hi, :, col + K:col + kp] = zgap[0:hi - lo]
            if lo > 0:
                il[0:lo, :, col:col + kp] = jnp.zeros((lo, B, kp), il.dtype)
            if hi < ho:
                il[hi:ho, :, col:col + kp] = jnp.zeros(
                    (ho - hi, B, kp), il.dtype)
        nc = g["wo"] * _CH
        acc = jnp.dot(il[0:ho, :, 0:k * kp].reshape(ho * B, k * kp),
                      ws[li][...], preferred_element_type=jnp.float32)
        a = jnp.maximum(acc + bs[li][...], 0.0)
        act = a.astype(jnp.bfloat16).reshape(ho, B, nc)
        prev_rows = ho

    ho4 = plan[-1]["ho"]
    y = jnp.dot(act[0], lw[0], preferred_element_type=jnp.float32)
    for h in range(1, ho4):
        y = y + jnp.dot(act[h], lw[h], preferred_element_type=jnp.float32)
    feat = jax.nn.sigmoid(y + lb[...])

    bh = B // 2
    d = jnp.abs(feat[0:bh] - feat[bh:B])
    o_ref[...] = (jnp.sum(d * ow[...], axis=1, keepdims=True)
                  + ob[...]).astype(o_ref.dtype)


def kernel(x1, x2, t0, t1, t2, t3, b0, b1, b2, b3, lin_w3, lin_b,
           out_w, out_b):
    n = x1.shape[0]
    plan = _geometry([t0.shape, t1.shape, t2.shape, t3.shape])
    g0 = plan[0]
    hidden = lin_w3.shape[-1]

    B = 64 if (2 * n) % 64 == 0 else 2 * n
    bh = B // 2
    nb = (2 * n) // B

    x1t = jnp.transpose(x1[:, 0, :, :].astype(jnp.bfloat16), (1, 0, 2))
    x2t = jnp.transpose(x2[:, 0, :, :].astype(jnp.bfloat16), (1, 0, 2))

    tws = []
    for li, (t, g) in enumerate(zip((t0, t1, t2, t3), plan)):
        K, kp = _slab_width(g, li)
        c = g["cin"]
        w = t[:, c * _PAD:c * _PAD + K, :]
        if kp > K:
            w = jnp.pad(w, ((0, 0), (0, kp - K), (0, 0)))
        tws.append(w.reshape(g["k"] * kp, t.shape[2]).astype(jnp.bfloat16))
    lwb = lin_w3.astype(jnp.bfloat16)
    ow_row = out_w.reshape(1, hidden)

    in_specs = [
        pl.BlockSpec((g0["hin"], bh, g0["win"]), lambda i: (0, i, 0)),
        pl.BlockSpec((g0["hin"], bh, g0["win"]), lambda i: (0, i, 0)),
    ]
    for w in tws:
        in_specs.append(pl.BlockSpec(w.shape, lambda i: (0, 0)))
    for b in (b0, b1, b2, b3):
        in_specs.append(pl.BlockSpec(b.shape, lambda i: (0, 0)))
    in_specs.append(pl.BlockSpec(lwb.shape, lambda i: (0, 0, 0)))
    in_specs.append(pl.BlockSpec(lin_b.shape, lambda i: (0, 0)))
    in_specs.append(pl.BlockSpec(ow_row.shape, lambda i: (0, 0)))
    in_specs.append(pl.BlockSpec(out_b.shape, lambda i: (0, 0)))

    max_ho = max(g["ho"] for g in plan)
    max_kw = max(g["k"] * _slab_width(g, li)[1]
                 for li, g in enumerate(plan))
    scratch = [pltpu.VMEM((max_ho, B, max_kw), jnp.bfloat16)]

    out = pl.pallas_call(
        functools.partial(_fused_kernel, plan=plan, batch=B),
        out_shape=jax.ShapeDtypeStruct((n, 1), jnp.float32),
        grid=(nb,),
        in_specs=in_specs,
        out_specs=pl.BlockSpec((bh, 1), lambda i: (i, 0)),
        scratch_shapes=scratch,
        compiler_params=pltpu.CompilerParams(
            dimension_semantics=("parallel",)),
    )(x1t, x2t, *tws, b0, b1, b2, b3, lwb, lin_b, ow_row, out_b)
    return out


# speedup vs baseline: 1.1053x; 1.1053x over previous
import functools

import jax
import jax.numpy as jnp
from jax.experimental import pallas as pl
from jax.experimental.pallas import tpu as pltpu

_CH = 8
_PAD = 1


def _geometry(t_shapes):
    plan = []
    cin = 1
    for (k, wpcin, wocout) in t_shapes:
        wp = wpcin // cin
        wo = wocout // _CH
        win = wp - 2 * _PAD
        ho = wp - k + 1
        assert ho == wo
        plan.append(dict(k=k, cin=cin, win=win, hin=win, wp=wp, hp=wp,
                         ho=ho, wo=wo))
        cin = _CH
    return plan


def _slab_width(g, li):
    K = g["win"] * g["cin"]
    if li == 0:
        return K, K
    return K, -(-K // 128) * 128


def _fused_kernel(x1_ref, x2_ref, w0, w1, w2, w3, b0, b1, b2, b3,
                  lw, lb, ow, ob, o_ref, il0, il1, il2, il3, *,
                  plan, batch):
    B = batch
    bh = B // 2
    ws = (w0, w1, w2, w3)
    bs = (b0, b1, b2, b3)
    ils = (il0, il1, il2, il3)

    def bounds(i, ho, prev_rows):
        return max(0, 1 - i), min(ho, prev_rows + 1 - i)

    def zero_fills(li):
        g = plan[li]
        k, ho = g["k"], g["ho"]
        K, kp = _slab_width(g, li)
        il = ils[li]
        prev = plan[0]["hin"] if li == 0 else plan[li - 1]["ho"]
        zgap = (jnp.zeros((ho, B, kp - K), il.dtype) if kp > K else None)
        for i in range(k):
            col = i * kp
            lo, hi = bounds(i, ho, prev)
            if zgap is not None:
                il[lo:hi, :, col + K:col + kp] = zgap[0:hi - lo]
            if lo > 0:
                il[0:lo, :, col:col + kp] = jnp.zeros((lo, B, kp), il.dtype)
            if hi < ho:
                il[hi:ho, :, col:col + kp] = jnp.zeros(
                    (ho - hi, B, kp), il.dtype)

    def scatter(li, act_h, a0, a1):
        g = plan[li]
        k, ho = g["k"], g["ho"]
        K, kp = _slab_width(g, li)
        il = ils[li]
        prev = plan[0]["hin"] if li == 0 else plan[li - 1]["ho"]
        for i in range(k):
            col = i * kp
            lo, hi = bounds(i, ho, prev)
            jlo, jhi = max(lo, a0 + 1 - i), min(hi, a1 + 1 - i)
            if jlo < jhi:
                il[jlo:jhi, :, col:col + K] = (
                    act_h[i + jlo - 1 - a0:i + jhi - 1 - a0])

    zero_fills(0)
    g0 = plan[0]
    K0, kp0 = _slab_width(g0, 0)
    for i in range(g0["k"]):
        col = i * kp0
        lo, hi = bounds(i, g0["ho"], g0["hin"])
        ils[0][lo:hi, 0:bh, col:col + K0] = x1_ref[i + lo - 1:i + hi - 1]
        ils[0][lo:hi, bh:B, col:col + K0] = x2_ref[i + lo - 1:i + hi - 1]

    y = lb[...]
    for li, g in enumerate(plan):
        k, ho, nc = g["k"], g["ho"], g["wo"] * _CH
        K, kp = _slab_width(g, li)
        il = ils[li]
        if li + 1 < len(plan):
            zero_fills(li + 1)
        h1 = ho // 2
        for a0, a1 in ((0, h1), (h1, ho)):
            acc = jnp.dot(il[a0:a1, :, 0:k * kp].reshape((a1 - a0) * B,
                                                         k * kp),
                          ws[li][...], preferred_element_type=jnp.float32)
            a_ = jnp.maximum(acc + bs[li][...], 0.0)
            act_h = a_.astype(jnp.bfloat16).reshape(a1 - a0, B, nc)
            if li + 1 < len(plan):
                scatter(li + 1, act_h, a0, a1)
            else:
                for j in range(a1 - a0):
                    y = y + jnp.dot(act_h[j], lw[a0 + j],
                                    preferred_element_type=jnp.float32)
    feat = jax.nn.sigmoid(y)

    bh = B // 2
    d = jnp.abs(feat[0:bh] - feat[bh:B])
    o_ref[...] = (jnp.sum(d * ow[...], axis=1, keepdims=True)
                  + ob[...]).astype(o_ref.dtype)


def kernel(x1, x2, t0, t1, t2, t3, b0, b1, b2, b3, lin_w3, lin_b,
           out_w, out_b):
    n = x1.shape[0]
    plan = _geometry([t0.shape, t1.shape, t2.shape, t3.shape])
    g0 = plan[0]
    hidden = lin_w3.shape[-1]

    B = 32 if (2 * n) % 32 == 0 else 2 * n
    bh = B // 2
    nb = (2 * n) // B

    x1t = jnp.transpose(x1[:, 0, :, :].astype(jnp.bfloat16), (1, 0, 2))
    x2t = jnp.transpose(x2[:, 0, :, :].astype(jnp.bfloat16), (1, 0, 2))

    tws = []
    for li, (t, g) in enumerate(zip((t0, t1, t2, t3), plan)):
        K, kp = _slab_width(g, li)
        c = g["cin"]
        w = t[:, c * _PAD:c * _PAD + K, :]
        if kp > K:
            w = jnp.pad(w, ((0, 0), (0, kp - K), (0, 0)))
        tws.append(w.reshape(g["k"] * kp, t.shape[2]).astype(jnp.bfloat16))
    lwb = lin_w3.astype(jnp.bfloat16)
    ow_row = out_w.reshape(1, hidden)

    in_specs = [
        pl.BlockSpec((g0["hin"], bh, g0["win"]), lambda i: (0, i, 0)),
        pl.BlockSpec((g0["hin"], bh, g0["win"]), lambda i: (0, i, 0)),
    ]
    for w in tws:
        in_specs.append(pl.BlockSpec(w.shape, lambda i: (0, 0)))
    for b in (b0, b1, b2, b3):
        in_specs.append(pl.BlockSpec(b.shape, lambda i: (0, 0)))
    in_specs.append(pl.BlockSpec(lwb.shape, lambda i: (0, 0, 0)))
    in_specs.append(pl.BlockSpec(lin_b.shape, lambda i: (0, 0)))
    in_specs.append(pl.BlockSpec(ow_row.shape, lambda i: (0, 0)))
    in_specs.append(pl.BlockSpec(out_b.shape, lambda i: (0, 0)))

    scratch = []
    for li, g in enumerate(plan):
        K, kp = _slab_width(g, li)
        scratch.append(pltpu.VMEM((g["ho"], B, g["k"] * kp), jnp.bfloat16))

    out = pl.pallas_call(
        functools.partial(_fused_kernel, plan=plan, batch=B),
        out_shape=jax.ShapeDtypeStruct((n, 1), jnp.float32),
        grid=(nb,),
        in_specs=in_specs,
        out_specs=pl.BlockSpec((bh, 1), lambda i: (i, 0)),
        scratch_shapes=scratch,
        compiler_params=pltpu.CompilerParams(
            dimension_semantics=("parallel",)),
    )(x1t, x2t, *tws, b0, b1, b2, b3, lwb, lin_b, ow_row, out_b)
    return out


# speedup vs baseline: 1.1589x; 1.0485x over previous
import functools

import jax
import jax.numpy as jnp
from jax.experimental import pallas as pl
from jax.experimental.pallas import tpu as pltpu

_CH = 8
_PAD = 1


def _geometry(t_shapes):
    plan = []
    cin = 1
    for (k, wpcin, wocout) in t_shapes:
        wp = wpcin // cin
        wo = wocout // _CH
        win = wp - 2 * _PAD
        ho = wp - k + 1
        assert ho == wo
        plan.append(dict(k=k, cin=cin, win=win, hin=win, wp=wp, hp=wp,
                         ho=ho, wo=wo))
        cin = _CH
    return plan


def _slab_width(g, li):
    K = g["win"] * g["cin"]
    if li == 0:
        return K, K
    kp64 = -(-K // 64) * 64
    kp128 = -(-K // 128) * 128
    k = g["k"]
    if -(-(k * kp64) // 256) < -(-(k * kp128) // 256):
        return K, kp64
    return K, kp128


def _fused_kernel(x1_ref, x2_ref, w0, w1, w2, w3, b0, b1, b2, b3,
                  lw, lb, ow, ob, o_ref, il0, il1, il2, il3, *,
                  plan, batch):
    B = batch
    bh = B // 2
    ws = (w0, w1, w2, w3)
    bs = (b0, b1, b2, b3)
    ils = (il0, il1, il2, il3)

    def bounds(i, ho, prev_rows):
        return max(0, 1 - i), min(ho, prev_rows + 1 - i)

    def zero_fills(li):
        g = plan[li]
        k, ho = g["k"], g["ho"]
        K, kp = _slab_width(g, li)
        il = ils[li]
        prev = plan[0]["hin"] if li == 0 else plan[li - 1]["ho"]
        zgap = (jnp.zeros((ho, B, kp - K), il.dtype) if kp > K else None)
        for i in range(k):
            col = i * kp
            lo, hi = bounds(i, ho, prev)
            if zgap is not None:
                il[lo:hi, :, col + K:col + kp] = zgap[0:hi - lo]
            if lo > 0:
                il[0:lo, :, col:col + kp] = jnp.zeros((lo, B, kp), il.dtype)
            if hi < ho:
                il[hi:ho, :, col:col + kp] = jnp.zeros(
                    (ho - hi, B, kp), il.dtype)

    def scatter(li, act_h, a0, a1):
        g = plan[li]
        k, ho = g["k"], g["ho"]
        K, kp = _slab_width(g, li)
        il = ils[li]
        prev = plan[0]["hin"] if li == 0 else plan[li - 1]["ho"]
        for i in range(k):
            col = i * kp
            lo, hi = bounds(i, ho, prev)
            jlo, jhi = max(lo, a0 + 1 - i), min(hi, a1 + 1 - i)
            if jlo < jhi:
                il[jlo:jhi, :, col:col + K] = (
                    act_h[i + jlo - 1 - a0:i + jhi - 1 - a0])

    zero_fills(0)
    g0 = plan[0]
    K0, kp0 = _slab_width(g0, 0)
    for i in range(g0["k"]):
        col = i * kp0
        lo, hi = bounds(i, g0["ho"], g0["hin"])
        ils[0][lo:hi, 0:bh, col:col + K0] = x1_ref[i + lo - 1:i + hi - 1]
        ils[0][lo:hi, bh:B, col:col + K0] = x2_ref[i + lo - 1:i + hi - 1]

    y = lb[...]
    for li, g in enumerate(plan):
        k, ho, nc = g["k"], g["ho"], g["wo"] * _CH
        K, kp = _slab_width(g, li)
        il = ils[li]
        if li + 1 < len(plan):
            zero_fills(li + 1)
        h1 = ho // 2
        for a0, a1 in ((0, h1), (h1, ho)):
            acc = jnp.dot(il[a0:a1, :, 0:k * kp].reshape((a1 - a0) * B,
                                                         k * kp),
                          ws[li][...], preferred_element_type=jnp.float32)
            a_ = jnp.maximum(acc + bs[li][...], 0.0)
            act_h = a_.astype(jnp.bfloat16).reshape(a1 - a0, B, nc)
            if li + 1 < len(plan):
                scatter(li + 1, act_h, a0, a1)
            else:
                for j in range(a1 - a0):
                    y = y + jnp.dot(act_h[j], lw[a0 + j],
                                    preferred_element_type=jnp.float32)
    feat = jax.nn.sigmoid(y)

    bh = B // 2
    d = jnp.abs(feat[0:bh] - feat[bh:B])
    o_ref[...] = (jnp.sum(d * ow[...], axis=1, keepdims=True)
                  + ob[...]).astype(o_ref.dtype)


def kernel(x1, x2, t0, t1, t2, t3, b0, b1, b2, b3, lin_w3, lin_b,
           out_w, out_b):
    n = x1.shape[0]
    plan = _geometry([t0.shape, t1.shape, t2.shape, t3.shape])
    g0 = plan[0]
    hidden = lin_w3.shape[-1]

    B = 32 if (2 * n) % 32 == 0 else 2 * n
    bh = B // 2
    nb = (2 * n) // B

    x1t = jnp.transpose(x1[:, 0, :, :].astype(jnp.bfloat16), (1, 0, 2))
    x2t = jnp.transpose(x2[:, 0, :, :].astype(jnp.bfloat16), (1, 0, 2))

    tws = []
    for li, (t, g) in enumerate(zip((t0, t1, t2, t3), plan)):
        K, kp = _slab_width(g, li)
        c = g["cin"]
        w = t[:, c * _PAD:c * _PAD + K, :]
        if kp > K:
            w = jnp.pad(w, ((0, 0), (0, kp - K), (0, 0)))
        tws.append(w.reshape(g["k"] * kp, t.shape[2]).astype(jnp.bfloat16))
    lwb = lin_w3.astype(jnp.bfloat16)
    ow_row = out_w.reshape(1, hidden)

    in_specs = [
        pl.BlockSpec((g0["hin"], bh, g0["win"]), lambda i: (0, i, 0)),
        pl.BlockSpec((g0["hin"], bh, g0["win"]), lambda i: (0, i, 0)),
    ]
    for w in tws:
        in_specs.append(pl.BlockSpec(w.shape, lambda i: (0, 0)))
    for b in (b0, b1, b2, b3):
        in_specs.append(pl.BlockSpec(b.shape, lambda i: (0, 0)))
    in_specs.append(pl.BlockSpec(lwb.shape, lambda i: (0, 0, 0)))
    in_specs.append(pl.BlockSpec(lin_b.shape, lambda i: (0, 0)))
    in_specs.append(pl.BlockSpec(ow_row.shape, lambda i: (0, 0)))
    in_specs.append(pl.BlockSpec(out_b.shape, lambda i: (0, 0)))

    scratch = []
    for li, g in enumerate(plan):
        K, kp = _slab_width(g, li)
        scratch.append(pltpu.VMEM((g["ho"], B, g["k"] * kp), jnp.bfloat16))

    out = pl.pallas_call(
        functools.partial(_fused_kernel, plan=plan, batch=B),
        out_shape=jax.ShapeDtypeStruct((n, 1), jnp.float32),
        grid=(nb,),
        in_specs=in_specs,
        out_specs=pl.BlockSpec((bh, 1), lambda i: (i, 0)),
        scratch_shapes=scratch,
        compiler_params=pltpu.CompilerParams(
            dimension_semantics=("parallel",)),
    )(x1t, x2t, *tws, b0, b1, b2, b3, lwb, lin_b, ow_row, out_b)
    return out


# speedup vs baseline: 1.2675x; 1.0937x over previous
import functools

import jax
import jax.numpy as jnp
from jax.experimental import pallas as pl
from jax.experimental.pallas import tpu as pltpu

_CH = 8
_PAD = 1


def _geometry(t_shapes):
    plan = []
    cin = 1
    for (k, wpcin, wocout) in t_shapes:
        wp = wpcin // cin
        wo = wocout // _CH
        win = wp - 2 * _PAD
        ho = wp - k + 1
        assert ho == wo
        plan.append(dict(k=k, cin=cin, win=win, hin=win, wp=wp, hp=wp,
                         ho=ho, wo=wo))
        cin = _CH
    return plan


def _slab_width(g, li):
    K = g["win"] * g["cin"]
    if li == 0:
        return K, K
    k = g["k"]
    kp64 = -(-K // 64) * 64
    kp128 = -(-K // 128) * 128
    tiles = lambda p: -(-(k * p) // 256)
    if tiles(kp128) == tiles(K):
        return K, kp128
    if tiles(kp64) == tiles(K):
        return K, kp64
    return K, K


def _fused_kernel(x1_ref, x2_ref, w0, w1, w2, w3, b0, b1, b2, b3,
                  lw, lb, ow, ob, o_ref, il0, il1, il2, il3, *,
                  plan, batch):
    B = batch
    bh = B // 2
    ws = (w0, w1, w2, w3)
    bs = (b0, b1, b2, b3)
    ils = (il0, il1, il2, il3)

    def bounds(i, ho, prev_rows):
        return max(0, 1 - i), min(ho, prev_rows + 1 - i)

    def zero_fills(li):
        g = plan[li]
        k, ho = g["k"], g["ho"]
        K, kp = _slab_width(g, li)
        il = ils[li]
        prev = plan[0]["hin"] if li == 0 else plan[li - 1]["ho"]
        zgap = (jnp.zeros((ho, B, kp - K), il.dtype) if kp > K else None)
        for i in range(k):
            col = i * kp
            lo, hi = bounds(i, ho, prev)
            if zgap is not None:
                il[lo:hi, :, col + K:col + kp] = zgap[0:hi - lo]
            if lo > 0:
                il[0:lo, :, col:col + kp] = jnp.zeros((lo, B, kp), il.dtype)
            if hi < ho:
                il[hi:ho, :, col:col + kp] = jnp.zeros(
                    (ho - hi, B, kp), il.dtype)

    def scatter(li, act_h, a0, a1):
        g = plan[li]
        k, ho = g["k"], g["ho"]
        K, kp = _slab_width(g, li)
        il = ils[li]
        prev = plan[0]["hin"] if li == 0 else plan[li - 1]["ho"]
        for i in range(k):
            col = i * kp
            lo, hi = bounds(i, ho, prev)
            jlo, jhi = max(lo, a0 + 1 - i), min(hi, a1 + 1 - i)
            if jlo < jhi:
                il[jlo:jhi, :, col:col + K] = (
                    act_h[i + jlo - 1 - a0:i + jhi - 1 - a0])

    zero_fills(0)
    g0 = plan[0]
    K0, kp0 = _slab_width(g0, 0)
    for i in range(g0["k"]):
        col = i * kp0
        lo, hi = bounds(i, g0["ho"], g0["hin"])
        ils[0][lo:hi, 0:bh, col:col + K0] = x1_ref[i + lo - 1:i + hi - 1]
        ils[0][lo:hi, bh:B, col:col + K0] = x2_ref[i + lo - 1:i + hi - 1]

    y = lb[...]
    for li, g in enumerate(plan):
        k, ho, nc = g["k"], g["ho"], g["wo"] * _CH
        K, kp = _slab_width(g, li)
        il = ils[li]
        if li + 1 < len(plan):
            zero_fills(li + 1)
        h1 = ho // 2
        for a0, a1 in ((0, h1), (h1, ho)):
            acc = jnp.dot(il[a0:a1, :, 0:k * kp].reshape((a1 - a0) * B,
                                                         k * kp),
                          ws[li][...], preferred_element_type=jnp.float32)
            a_ = jnp.maximum(acc + bs[li][...], 0.0)
            act_h = a_.astype(jnp.bfloat16).reshape(a1 - a0, B, nc)
            if li + 1 < len(plan):
                scatter(li + 1, act_h, a0, a1)
            else:
                for j in range(a1 - a0):
                    y = y + jnp.dot(act_h[j], lw[a0 + j],
                                    preferred_element_type=jnp.float32)
    feat = jax.nn.sigmoid(y)

    bh = B // 2
    d = jnp.abs(feat[0:bh] - feat[bh:B])
    o_ref[...] = (jnp.sum(d * ow[...], axis=1, keepdims=True)
                  + ob[...]).astype(o_ref.dtype)


def kernel(x1, x2, t0, t1, t2, t3, b0, b1, b2, b3, lin_w3, lin_b,
           out_w, out_b):
    n = x1.shape[0]
    plan = _geometry([t0.shape, t1.shape, t2.shape, t3.shape])
    g0 = plan[0]
    hidden = lin_w3.shape[-1]

    B = 32 if (2 * n) % 32 == 0 else 2 * n
    bh = B // 2
    nb = (2 * n) // B

    x1t = jnp.transpose(x1[:, 0, :, :].astype(jnp.bfloat16), (1, 0, 2))
    x2t = jnp.transpose(x2[:, 0, :, :].astype(jnp.bfloat16), (1, 0, 2))

    tws = []
    for li, (t, g) in enumerate(zip((t0, t1, t2, t3), plan)):
        K, kp = _slab_width(g, li)
        c = g["cin"]
        w = t[:, c * _PAD:c * _PAD + K, :]
        if kp > K:
            w = jnp.pad(w, ((0, 0), (0, kp - K), (0, 0)))
        tws.append(w.reshape(g["k"] * kp, t.shape[2]).astype(jnp.bfloat16))
    lwb = lin_w3.astype(jnp.bfloat16)
    ow_row = out_w.reshape(1, hidden)

    in_specs = [
        pl.BlockSpec((g0["hin"], bh, g0["win"]), lambda i: (0, i, 0)),
        pl.BlockSpec((g0["hin"], bh, g0["win"]), lambda i: (0, i, 0)),
    ]
    for w in tws:
        in_specs.append(pl.BlockSpec(w.shape, lambda i: (0, 0)))
    for b in (b0, b1, b2, b3):
        in_specs.append(pl.BlockSpec(b.shape, lambda i: (0, 0)))
    in_specs.append(pl.BlockSpec(lwb.shape, lambda i: (0, 0, 0)))
    in_specs.append(pl.BlockSpec(lin_b.shape, lambda i: (0, 0)))
    in_specs.append(pl.BlockSpec(ow_row.shape, lambda i: (0, 0)))
    in_specs.append(pl.BlockSpec(out_b.shape, lambda i: (0, 0)))

    scratch = []
    for li, g in enumerate(plan):
        K, kp = _slab_width(g, li)
        scratch.append(pltpu.VMEM((g["ho"], B, g["k"] * kp), jnp.bfloat16))

    out = pl.pallas_call(
        functools.partial(_fused_kernel, plan=plan, batch=B),
        out_shape=jax.ShapeDtypeStruct((n, 1), jnp.float32),
        grid=(nb,),
        in_specs=in_specs,
        out_specs=pl.BlockSpec((bh, 1), lambda i: (i, 0)),
        scratch_shapes=scratch,
        compiler_params=pltpu.CompilerParams(
            dimension_semantics=("parallel",)),
    )(x1t, x2t, *tws, b0, b1, b2, b3, lwb, lin_b, ow_row, out_b)
    return out
